# baseline (speedup 1.0000x reference)
import math

import numpy as np
import jax
import jax.numpy as jnp
from jax.experimental import pallas as pl
from jax.experimental.pallas import tpu as pltpu
from jax.scipy.special import gammaln

_N_PART = 16384
_T = 256
_MAX_TAU = 25
_BLOCK = 2048
_HALF_LOG2PI = 0.5 * math.log(2.0 * math.pi)


_GBN = 1024
_GSTRIP = 64

_ONE_BITS = np.uint32(np.float32(1.0).view(np.uint32))
_U_LO = np.float32(np.nextafter(np.float32(-1.0), np.float32(0.0)))
_SQRT2 = np.float32(np.sqrt(2.0))


def _rotl(x, r):
    return (x << np.uint32(r)) | (x >> np.uint32(32 - r))


def _tf_hash(k1, k2, c1, c2):
    ks0, ks1 = k1, k2
    ks2 = k1 ^ k2 ^ np.uint32(0x1BD11BDA)
    x0 = c1 + ks0
    x1 = c2 + ks1

    def rounds(x0, x1, rots):
        for r in rots:
            x0 = x0 + x1
            x1 = _rotl(x1, r)
            x1 = x0 ^ x1
        return x0, x1

    r0 = (13, 15, 26, 6)
    r1 = (17, 29, 16, 24)
    x0, x1 = rounds(x0, x1, r0)
    x0, x1 = x0 + ks1, x1 + ks2 + np.uint32(1)
    x0, x1 = rounds(x0, x1, r1)
    x0, x1 = x0 + ks2, x1 + ks0 + np.uint32(2)
    x0, x1 = rounds(x0, x1, r0)
    x0, x1 = x0 + ks0, x1 + ks1 + np.uint32(3)
    x0, x1 = rounds(x0, x1, r1)
    x0, x1 = x0 + ks1, x1 + ks2 + np.uint32(4)
    x0, x1 = rounds(x0, x1, r0)
    x0, x1 = x0 + ks2, x1 + ks0 + np.uint32(5)
    return x0, x1


def _bits_to_float01(bits):
    fb = (bits >> np.uint32(9)) | _ONE_BITS
    f = jax.lax.bitcast_convert_type(fb, jnp.float32) - jnp.float32(1.0)
    return jnp.maximum(jnp.float32(0.0),
                       f * jnp.float32(1.0) + jnp.float32(0.0))


def _bits_to_normal(bits):
    fb = (bits >> np.uint32(9)) | _ONE_BITS
    f = jax.lax.bitcast_convert_type(fb, jnp.float32) - jnp.float32(1.0)
    u = jnp.maximum(_U_LO, f * (np.float32(1.0) - _U_LO) + _U_LO)
    return _SQRT2 * jax.lax.erf_inv(u)


def _draw_pair(k1, k2, c):
    zero = jnp.zeros_like(k1)
    one_u = jnp.ones_like(k1)
    p1b, p2b = _tf_hash(k1, k2, zero, one_u)
    nb1, nb2 = _tf_hash(p1b, p2b, zero, zero)
    x = _bits_to_normal(nb1 ^ nb2)
    v = jnp.float32(1.0) + x * c
    return x, v


def _attempt(k1, k2, c, active):
    zero = jnp.zeros_like(k1)
    one_u = jnp.ones_like(k1)
    f32 = jnp.float32
    xk1, xk2 = _tf_hash(k1, k2, zero, one_u)
    uk1, uk2 = _tf_hash(k1, k2, zero, one_u + one_u)
    x, v = _draw_pair(xk1, xk2, c)

    def in_cond(ist):
        return jnp.any(ist[4] != 0)

    def in_body(ist):
        pk1, pk2, x, v, iact_i = ist
        iact = iact_i != 0
        a1, a2 = _tf_hash(pk1, pk2, zero, zero)
        x2, v2 = _draw_pair(a1, a2, c)
        x = jnp.where(iact, x2, x)
        v = jnp.where(iact, v2, v)
        pk1 = jnp.where(iact, a1, pk1)
        pk2 = jnp.where(iact, a2, pk2)
        iact_i = jnp.where(v <= f32(0.0), iact_i, jnp.int32(0))
        return pk1, pk2, x, v, iact_i

    iact0 = jnp.where(active & (v <= f32(0.0)), jnp.int32(1), jnp.int32(0))
    _, _, x, v, _ = jax.lax.while_loop(
        in_cond, in_body, (xk1, xk2, x, v, iact0))

    X = x * x
    V = (v * v) * v
    ub1, ub2 = _tf_hash(uk1, uk2, zero, zero)
    U = _bits_to_float01(ub1 ^ ub2)
    return X, V, U


def _gamma_strip(key0_1, key0_2, alpha, alpha_s):
    zero = jnp.zeros_like(key0_1)
    one_u = jnp.ones_like(key0_1)
    f32 = jnp.float32
    alpha_eff = jnp.where(alpha >= f32(1.0), alpha, alpha + f32(1.0))
    d = alpha_eff - f32(1.0 / 3.0)
    c = f32(1.0 / 3.0) / jnp.sqrt(d)
    shp = key0_1.shape

    o1a, o2a = _tf_hash(key0_1, key0_2, zero, zero)

    def rej_of(X, V, U):
        return (U >= f32(1.0) - f32(0.0331) * (X * X)) & (
            jnp.log(U) >= X * f32(0.5)
            + d * ((f32(1.0) - V) + jnp.log(V)))

    all_on = jnp.ones(shp, jnp.bool_)
    X, V, U = _attempt(o1a, o2a, c, all_on)
    active0 = jnp.where(rej_of(X, V, U), jnp.int32(1), jnp.int32(0))

    def outer_cond(st):
        return jnp.any(st[5] != 0)

    def outer_body(st):
        k1, k2, X, V, U, active_i = st
        active = active_i != 0
        n1a, n2a = _tf_hash(k1, k2, zero, zero)
        k1 = jnp.where(active, n1a, k1)
        k2 = jnp.where(active, n2a, k2)
        Xn, Vn, Un = _attempt(k1, k2, c, active)
        X = jnp.where(active, Xn, X)
        V = jnp.where(active, Vn, V)
        U = jnp.where(active, Un, U)
        active_i = jnp.where(rej_of(X, V, U), active_i, jnp.int32(0))
        return k1, k2, X, V, U, active_i

    st0 = (o1a, o2a, X, V, U, active0)
    _, _, _, V, _, _ = jax.lax.while_loop(outer_cond, outer_body, st0)

    def boost_one(_):
        return jnp.ones(shp, f32)

    def boost_general(_):
        sf1, sf2 = _tf_hash(key0_1, key0_2, zero, one_u)
        fb1, fb2 = _tf_hash(sf1, sf2, zero, zero)
        samples = f32(1.0) - _bits_to_float01(fb1 ^ fb2)
        return jnp.exp(jnp.log(samples) * (f32(1.0) / alpha))

    boost = jax.lax.cond(alpha_s >= f32(1.0), boost_one, boost_general, 0)
    return (d * V) * boost


def _gamma_kernel(keys_ref, alphas_ref, out_ref):
    a = pl.program_id(0)
    b = pl.program_id(1)
    k1s = jnp.where(a == 0, keys_ref[0], keys_ref[2])
    k2s = jnp.where(a == 0, keys_ref[1], keys_ref[3])
    alpha_s = jnp.where(a == 0, alphas_ref[0], alphas_ref[1])

    T = out_ref.shape[2]
    row_ids = jax.lax.broadcasted_iota(jnp.int32, (_GSTRIP, T), 0)
    col_ids = jax.lax.broadcasted_iota(jnp.int32, (_GSTRIP, T), 1)
    kb1 = jnp.full((_GSTRIP, T), k1s, jnp.uint32)
    kb2 = jnp.full((_GSTRIP, T), k2s, jnp.uint32)
    alpha = jnp.full((_GSTRIP, T), alpha_s, jnp.float32)

    def strip_body(r, _):
        base_row = b * _GBN + r * _GSTRIP
        s = ((base_row + row_ids) * T + col_ids).astype(jnp.uint32)
        key0_1, key0_2 = _tf_hash(kb1, kb2, jnp.zeros_like(s), s)
        vals = _gamma_strip(key0_1, key0_2, alpha, alpha_s)
        out_ref[0, pl.ds(r * _GSTRIP, _GSTRIP), :] = vals / alpha
        return 0

    jax.lax.fori_loop(0, _GBN // _GSTRIP, strip_body, 0)


def _gamma_pair(k2_key, k3_key, a_p, a_d, T):
    keys = jnp.concatenate([jax.random.key_data(k2_key),
                            jax.random.key_data(k3_key)]).astype(jnp.uint32)
    alphas = jnp.stack([a_p, a_d]).astype(jnp.float32)
    nb = _N_PART // _GBN
    out = pl.pallas_call(
        _gamma_kernel,
        grid=(2, nb),
        in_specs=[
            pl.BlockSpec(memory_space=pltpu.SMEM),
            pl.BlockSpec(memory_space=pltpu.SMEM),
        ],
        out_specs=pl.BlockSpec((1, _GBN, T), lambda a, b: (a, b, 0)),
        out_shape=jax.ShapeDtypeStruct((2, _N_PART, T), jnp.float32),
        compiler_params=pltpu.CompilerParams(
            dimension_semantics=("arbitrary", "arbitrary")),
    )(keys, alphas)
    return out[0], out[1]


_WIN = 32
_CHUNK = 8


def _recurrence_kernel(scal_ref, sig0_ref, tau_ref, lp0_ref, et_ref, ep_ref,
                       el_ref, zsT_ref, lp_ref):
    P = scal_ref[0]
    N_0 = scal_ref[1]
    delta = scal_ref[2]
    sigma = scal_ref[3]
    log_sigma = scal_ref[4]
    two_sig2 = scal_ref[5]
    a_p = scal_ref[6]
    c_p = scal_ref[7]
    a_d = scal_ref[8]
    c_d = scal_ref[9]

    B = zsT_ref.shape[1]
    T = et_ref.shape[0]

    w_rows = [jnp.zeros((1, B), jnp.float32) for _ in range(_WIN - _MAX_TAU)]
    w_rows += [jnp.full((1, B), sig0_ref[j]) for j in range(_MAX_TAU)]
    W0 = jnp.concatenate(w_rows, axis=0)

    zsT_ref[0:_WIN, :] = W0

    tau_blk = tau_ref[0]
    row_ids = jax.lax.broadcasted_iota(jnp.int32, (_WIN, B), 0)
    onehot = (row_ids == (_WIN - tau_blk)).astype(jnp.float32)

    et_all = et_ref[:, :]
    ep_all = ep_ref[:, :]
    lp_et = jnp.sum(c_p + (a_p - 1.0) * jnp.log(et_all) - a_p * et_all,
                    axis=0, keepdims=True)
    lp_ep = jnp.sum(c_d + (a_d - 1.0) * jnp.log(ep_all) - a_d * ep_all,
                    axis=0, keepdims=True)
    lp0 = (lp0_ref[0] + lp_et) + lp_ep

    z0 = jnp.full((1, B), sig0_ref[_MAX_TAU - 1])

    def chunk_body(c, carry):
        W, z, lp = carry
        base = pl.multiple_of(c * _CHUNK, _CHUNK)
        et8 = et_ref[pl.ds(base, _CHUNK), :]
        ep8 = ep_ref[pl.ds(base, _CHUNK), :]
        el8 = el_ref[pl.ds(base, _CHUNK), :]
        z_news = []
        for j in range(_CHUNK):
            ztmtau = jnp.sum(W * onehot, axis=0, keepdims=True)
            et_t = et8[j:j + 1, :]
            ep_t = ep8[j:j + 1, :]
            e_t = el8[j:j + 1, :]
            z_mean = (P * ztmtau * jnp.exp(-ztmtau / N_0) * et_t
                      + z * jnp.exp(-delta * ep_t))
            mu = jnp.log(z_mean)
            z_new = jnp.exp(mu + sigma * e_t)
            logz = jnp.log(z_new)
            lp = lp + (-logz - log_sigma - _HALF_LOG2PI
                       - (logz - mu) ** 2 / two_sig2)
            W = jnp.concatenate([W[1:, :], z_new], axis=0)
            z = z_new
            z_news.append(z_new)
        zsT_ref[pl.ds(_WIN + base, _CHUNK), :] = jnp.concatenate(
            z_news, axis=0)
        return W, z, lp

    _, _, lp = jax.lax.fori_loop(0, T // _CHUNK, chunk_body, (W0, z0, lp0))
    lp_ref[0] = lp


def kernel(N, I, P_log, N_0_log, z_0, s_d_log, s_p_log, tau_logits, delta_log,
           noise_std_log, rand_std_log):
    T = I.shape[0]
    max_tau = tau_logits.shape[0]
    N_static = _N_PART
    P = jnp.exp(P_log)
    N_0 = jnp.exp(N_0_log)
    s_d = jnp.exp(s_d_log)
    s_p = jnp.exp(s_p_log)
    delta = jnp.exp(delta_log)
    sigma = jnp.exp(rand_std_log)

    key = jax.random.key(42)
    k1, k2, k3, k4 = jax.random.split(key, 4)
    tau0 = jax.random.categorical(k1, tau_logits, shape=(N_static,))
    lp0 = jax.nn.log_softmax(tau_logits)[tau0]
    tau = tau0 + 1 + (N - N)
    a_p = 1.0 / (s_p ** 2)
    a_d = 1.0 / (s_d ** 2)
    et, epsilont = _gamma_pair(k2, k3, a_p, a_d, T)
    eps_ln = jax.random.normal(k4, (N_static, T), dtype=jnp.float32)

    scal = jnp.stack([
        P, N_0, delta, sigma, jnp.log(sigma), 2.0 * sigma ** 2,
        a_p, a_p * jnp.log(a_p) - gammaln(a_p),
        a_d, a_d * jnp.log(a_d) - gammaln(a_d),
    ]).astype(jnp.float32)
    sig0 = jax.nn.sigmoid(z_0).astype(jnp.float32)

    nb = N_static // _BLOCK
    tau_r = tau.astype(jnp.int32).reshape(nb, 1, _BLOCK)
    lp0_r = lp0.reshape(nb, 1, _BLOCK)
    et_T = et.T
    ep_T = epsilont.T
    el_T = eps_ln.T

    zsT, lp_out = pl.pallas_call(
        _recurrence_kernel,
        grid=(nb,),
        in_specs=[
            pl.BlockSpec(memory_space=pltpu.SMEM),
            pl.BlockSpec(memory_space=pltpu.SMEM),
            pl.BlockSpec((1, 1, _BLOCK), lambda i: (i, 0, 0)),
            pl.BlockSpec((1, 1, _BLOCK), lambda i: (i, 0, 0)),
            pl.BlockSpec((T, _BLOCK), lambda i: (0, i)),
            pl.BlockSpec((T, _BLOCK), lambda i: (0, i)),
            pl.BlockSpec((T, _BLOCK), lambda i: (0, i)),
        ],
        out_specs=[
            pl.BlockSpec((_WIN - _MAX_TAU + max_tau + T, _BLOCK),
                         lambda i: (0, i)),
            pl.BlockSpec((1, 1, _BLOCK), lambda i: (i, 0, 0)),
        ],
        out_shape=[
            jax.ShapeDtypeStruct((_WIN - _MAX_TAU + max_tau + T, N_static),
                                 jnp.float32),
            jax.ShapeDtypeStruct((nb, 1, _BLOCK), jnp.float32),
        ],
        compiler_params=pltpu.CompilerParams(
            dimension_semantics=("arbitrary",)),
    )(scal, sig0, tau_r, lp0_r, et_T, ep_T, el_T)

    zs = zsT[_WIN - _MAX_TAU:, :].T
    xs = jax.lax.stop_gradient(zs[:, -T:])
    logp0 = lp_out.reshape(N_static)
    nat = jnp.zeros((N_static, T * 2), dtype=xs.dtype)
    return (tau, et, epsilont, zs, xs, -logp0, nat)


# speedup vs baseline: 1.0429x; 1.0429x over previous
import math

import numpy as np
import jax
import jax.numpy as jnp
from jax.experimental import pallas as pl
from jax.experimental.pallas import tpu as pltpu
from jax.scipy.special import gammaln

_N_PART = 16384
_T = 256
_MAX_TAU = 25
_BLOCK = 2048
_HALF_LOG2PI = 0.5 * math.log(2.0 * math.pi)


_GBN = 1024
_GSTRIP = 64

_ONE_BITS = np.uint32(np.float32(1.0).view(np.uint32))
_U_LO = np.float32(np.nextafter(np.float32(-1.0), np.float32(0.0)))
_SQRT2 = np.float32(np.sqrt(2.0))


def _rotl(x, r):
    return (x << np.uint32(r)) | (x >> np.uint32(32 - r))


def _tf_hash(k1, k2, c1, c2):
    ks0, ks1 = k1, k2
    ks2 = k1 ^ k2 ^ np.uint32(0x1BD11BDA)
    x0 = c1 + ks0
    x1 = c2 + ks1

    def rounds(x0, x1, rots):
        for r in rots:
            x0 = x0 + x1
            x1 = _rotl(x1, r)
            x1 = x0 ^ x1
        return x0, x1

    r0 = (13, 15, 26, 6)
    r1 = (17, 29, 16, 24)
    x0, x1 = rounds(x0, x1, r0)
    x0, x1 = x0 + ks1, x1 + ks2 + np.uint32(1)
    x0, x1 = rounds(x0, x1, r1)
    x0, x1 = x0 + ks2, x1 + ks0 + np.uint32(2)
    x0, x1 = rounds(x0, x1, r0)
    x0, x1 = x0 + ks0, x1 + ks1 + np.uint32(3)
    x0, x1 = rounds(x0, x1, r1)
    x0, x1 = x0 + ks1, x1 + ks2 + np.uint32(4)
    x0, x1 = rounds(x0, x1, r0)
    x0, x1 = x0 + ks2, x1 + ks0 + np.uint32(5)
    return x0, x1


def _bits_to_float01(bits):
    fb = (bits >> np.uint32(9)) | _ONE_BITS
    f = jax.lax.bitcast_convert_type(fb, jnp.float32) - jnp.float32(1.0)
    return jnp.maximum(jnp.float32(0.0),
                       f * jnp.float32(1.0) + jnp.float32(0.0))


def _bits_to_normal(bits):
    fb = (bits >> np.uint32(9)) | _ONE_BITS
    f = jax.lax.bitcast_convert_type(fb, jnp.float32) - jnp.float32(1.0)
    u = jnp.maximum(_U_LO, f * (np.float32(1.0) - _U_LO) + _U_LO)
    return _SQRT2 * jax.lax.erf_inv(u)


def _draw_pair(k1, k2, c):
    zero = jnp.zeros_like(k1)
    one_u = jnp.ones_like(k1)
    p1b, p2b = _tf_hash(k1, k2, zero, one_u)
    nb1, nb2 = _tf_hash(p1b, p2b, zero, zero)
    x = _bits_to_normal(nb1 ^ nb2)
    v = jnp.float32(1.0) + x * c
    return x, v


def _attempt(k1, k2, c, active):
    zero = jnp.zeros_like(k1)
    one_u = jnp.ones_like(k1)
    f32 = jnp.float32
    xk1, xk2 = _tf_hash(k1, k2, zero, one_u)
    uk1, uk2 = _tf_hash(k1, k2, zero, one_u + one_u)
    x, v = _draw_pair(xk1, xk2, c)

    def in_cond(ist):
        return jnp.any(ist[4] != 0)

    def in_body(ist):
        pk1, pk2, x, v, iact_i = ist
        iact = iact_i != 0
        a1, a2 = _tf_hash(pk1, pk2, zero, zero)
        x2, v2 = _draw_pair(a1, a2, c)
        x = jnp.where(iact, x2, x)
        v = jnp.where(iact, v2, v)
        pk1 = jnp.where(iact, a1, pk1)
        pk2 = jnp.where(iact, a2, pk2)
        iact_i = jnp.where(v <= f32(0.0), iact_i, jnp.int32(0))
        return pk1, pk2, x, v, iact_i

    iact0 = jnp.where(active & (v <= f32(0.0)), jnp.int32(1), jnp.int32(0))
    _, _, x, v, _ = jax.lax.while_loop(
        in_cond, in_body, (xk1, xk2, x, v, iact0))

    X = x * x
    V = (v * v) * v
    ub1, ub2 = _tf_hash(uk1, uk2, zero, zero)
    U = _bits_to_float01(ub1 ^ ub2)
    return X, V, U


def _gamma_strip(key0_1, key0_2, alpha, alpha_s):
    zero = jnp.zeros_like(key0_1)
    one_u = jnp.ones_like(key0_1)
    f32 = jnp.float32
    alpha_eff = jnp.where(alpha >= f32(1.0), alpha, alpha + f32(1.0))
    d = alpha_eff - f32(1.0 / 3.0)
    c = f32(1.0 / 3.0) / jnp.sqrt(d)
    shp = key0_1.shape

    o1a, o2a = _tf_hash(key0_1, key0_2, zero, zero)

    def rej_of(X, V, U):
        return (U >= f32(1.0) - f32(0.0331) * (X * X)) & (
            jnp.log(U) >= X * f32(0.5)
            + d * ((f32(1.0) - V) + jnp.log(V)))

    all_on = jnp.ones(shp, jnp.bool_)
    X, V, U = _attempt(o1a, o2a, c, all_on)
    active0 = jnp.where(rej_of(X, V, U), jnp.int32(1), jnp.int32(0))

    def outer_cond(st):
        return jnp.any(st[5] != 0)

    def outer_body(st):
        k1, k2, X, V, U, active_i = st
        active = active_i != 0
        n1a, n2a = _tf_hash(k1, k2, zero, zero)
        k1 = jnp.where(active, n1a, k1)
        k2 = jnp.where(active, n2a, k2)
        Xn, Vn, Un = _attempt(k1, k2, c, active)
        X = jnp.where(active, Xn, X)
        V = jnp.where(active, Vn, V)
        U = jnp.where(active, Un, U)
        active_i = jnp.where(rej_of(X, V, U), active_i, jnp.int32(0))
        return k1, k2, X, V, U, active_i

    st0 = (o1a, o2a, X, V, U, active0)
    _, _, _, V, _, _ = jax.lax.while_loop(outer_cond, outer_body, st0)

    def boost_one(_):
        return jnp.ones(shp, f32)

    def boost_general(_):
        sf1, sf2 = _tf_hash(key0_1, key0_2, zero, one_u)
        fb1, fb2 = _tf_hash(sf1, sf2, zero, zero)
        samples = f32(1.0) - _bits_to_float01(fb1 ^ fb2)
        return jnp.exp(jnp.log(samples) * (f32(1.0) / alpha))

    boost = jax.lax.cond(alpha_s >= f32(1.0), boost_one, boost_general, 0)
    return (d * V) * boost


def _gamma_kernel(keys_ref, alphas_ref, out_ref):
    a = pl.program_id(0)
    b = pl.program_id(1)
    k1s = jnp.where(a == 0, keys_ref[0], keys_ref[2])
    k2s = jnp.where(a == 0, keys_ref[1], keys_ref[3])
    alpha_s = jnp.where(a == 0, alphas_ref[0], alphas_ref[1])

    T = out_ref.shape[2]
    row_ids = jax.lax.broadcasted_iota(jnp.int32, (_GSTRIP, T), 0)
    col_ids = jax.lax.broadcasted_iota(jnp.int32, (_GSTRIP, T), 1)
    kb1 = jnp.full((_GSTRIP, T), k1s, jnp.uint32)
    kb2 = jnp.full((_GSTRIP, T), k2s, jnp.uint32)
    alpha = jnp.full((_GSTRIP, T), alpha_s, jnp.float32)

    def strip_body(r, _):
        base_row = b * _GBN + r * _GSTRIP
        s = ((base_row + row_ids) * T + col_ids).astype(jnp.uint32)
        key0_1, key0_2 = _tf_hash(kb1, kb2, jnp.zeros_like(s), s)
        vals = _gamma_strip(key0_1, key0_2, alpha, alpha_s)
        out_ref[0, pl.ds(r * _GSTRIP, _GSTRIP), :] = vals / alpha
        return 0

    jax.lax.fori_loop(0, _GBN // _GSTRIP, strip_body, 0)


def _gamma_pair(k2_key, k3_key, a_p, a_d, T):
    keys = jnp.concatenate([jax.random.key_data(k2_key),
                            jax.random.key_data(k3_key)]).astype(jnp.uint32)
    alphas = jnp.stack([a_p, a_d]).astype(jnp.float32)
    nb = _N_PART // _GBN
    out = pl.pallas_call(
        _gamma_kernel,
        grid=(2, nb),
        in_specs=[
            pl.BlockSpec(memory_space=pltpu.SMEM),
            pl.BlockSpec(memory_space=pltpu.SMEM),
        ],
        out_specs=pl.BlockSpec((1, _GBN, T), lambda a, b: (a, b, 0)),
        out_shape=jax.ShapeDtypeStruct((2, _N_PART, T), jnp.float32),
        compiler_params=pltpu.CompilerParams(
            dimension_semantics=("arbitrary", "arbitrary")),
    )(keys, alphas)
    return out[0], out[1]


_WIN = 32
_CHUNK = 8


def _recurrence_kernel(scal_ref, sig0_ref, tau_ref, lp0_ref, et_ref, ep_ref,
                       el_ref, zsT_ref, lp_ref):
    P = scal_ref[0]
    N_0 = scal_ref[1]
    delta = scal_ref[2]
    sigma = scal_ref[3]
    log_sigma = scal_ref[4]
    two_sig2 = scal_ref[5]
    a_p = scal_ref[6]
    c_p = scal_ref[7]
    a_d = scal_ref[8]
    c_d = scal_ref[9]

    B = zsT_ref.shape[1]
    T = et_ref.shape[0]

    w_rows = [jnp.zeros((1, B), jnp.float32) for _ in range(_WIN - _MAX_TAU)]
    w_rows += [jnp.full((1, B), sig0_ref[j]) for j in range(_MAX_TAU)]
    W0 = jnp.concatenate(w_rows, axis=0)

    zsT_ref[0:_WIN, :] = W0

    tau_blk = tau_ref[0]
    tap_idx = _WIN - tau_blk
    tap_grp = tap_idx >> 3
    tap_row = tap_idx & 7

    et_all = et_ref[:, :]
    ep_all = ep_ref[:, :]
    lp_et = jnp.sum(c_p + (a_p - 1.0) * jnp.log(et_all) - a_p * et_all,
                    axis=0, keepdims=True)
    lp_ep = jnp.sum(c_d + (a_d - 1.0) * jnp.log(ep_all) - a_d * ep_all,
                    axis=0, keepdims=True)
    lp0 = (lp0_ref[0] + lp_et) + lp_ep

    z0 = jnp.full((1, B), sig0_ref[_MAX_TAU - 1])

    def chunk_body(c, carry):
        W, z, lp = carry
        base = pl.multiple_of(c * _CHUNK, _CHUNK)
        et8 = et_ref[pl.ds(base, _CHUNK), :]
        ep8 = ep_ref[pl.ds(base, _CHUNK), :]
        el8 = el_ref[pl.ds(base, _CHUNK), :]
        z_news = []
        for j in range(_CHUNK):
            gats = [jnp.take_along_axis(W[8 * g:8 * g + 8, :], tap_row,
                                        axis=0) for g in range(4)]
            ztmtau = jnp.where(
                tap_grp == 0, gats[0],
                jnp.where(tap_grp == 1, gats[1],
                          jnp.where(tap_grp == 2, gats[2], gats[3])))
            et_t = et8[j:j + 1, :]
            ep_t = ep8[j:j + 1, :]
            e_t = el8[j:j + 1, :]
            z_mean = (P * ztmtau * jnp.exp(-ztmtau / N_0) * et_t
                      + z * jnp.exp(-delta * ep_t))
            mu = jnp.log(z_mean)
            z_new = jnp.exp(mu + sigma * e_t)
            logz = jnp.log(z_new)
            lp = lp + (-logz - log_sigma - _HALF_LOG2PI
                       - (logz - mu) ** 2 / two_sig2)
            W = jnp.concatenate([W[1:, :], z_new], axis=0)
            z = z_new
            z_news.append(z_new)
        zsT_ref[pl.ds(_WIN + base, _CHUNK), :] = jnp.concatenate(
            z_news, axis=0)
        return W, z, lp

    _, _, lp = jax.lax.fori_loop(0, T // _CHUNK, chunk_body, (W0, z0, lp0))
    lp_ref[0] = lp


def kernel(N, I, P_log, N_0_log, z_0, s_d_log, s_p_log, tau_logits, delta_log,
           noise_std_log, rand_std_log):
    T = I.shape[0]
    max_tau = tau_logits.shape[0]
    N_static = _N_PART
    P = jnp.exp(P_log)
    N_0 = jnp.exp(N_0_log)
    s_d = jnp.exp(s_d_log)
    s_p = jnp.exp(s_p_log)
    delta = jnp.exp(delta_log)
    sigma = jnp.exp(rand_std_log)

    key = jax.random.key(42)
    k1, k2, k3, k4 = jax.random.split(key, 4)
    tau0 = jax.random.categorical(k1, tau_logits, shape=(N_static,))
    lp0 = jax.nn.log_softmax(tau_logits)[tau0]
    tau = tau0 + 1 + (N - N)
    a_p = 1.0 / (s_p ** 2)
    a_d = 1.0 / (s_d ** 2)
    et, epsilont = _gamma_pair(k2, k3, a_p, a_d, T)
    eps_ln = jax.random.normal(k4, (N_static, T), dtype=jnp.float32)

    scal = jnp.stack([
        P, N_0, delta, sigma, jnp.log(sigma), 2.0 * sigma ** 2,
        a_p, a_p * jnp.log(a_p) - gammaln(a_p),
        a_d, a_d * jnp.log(a_d) - gammaln(a_d),
    ]).astype(jnp.float32)
    sig0 = jax.nn.sigmoid(z_0).astype(jnp.float32)

    nb = N_static // _BLOCK
    tau_r = tau.astype(jnp.int32).reshape(nb, 1, _BLOCK)
    lp0_r = lp0.reshape(nb, 1, _BLOCK)
    et_T = et.T
    ep_T = epsilont.T
    el_T = eps_ln.T

    zsT, lp_out = pl.pallas_call(
        _recurrence_kernel,
        grid=(nb,),
        in_specs=[
            pl.BlockSpec(memory_space=pltpu.SMEM),
            pl.BlockSpec(memory_space=pltpu.SMEM),
            pl.BlockSpec((1, 1, _BLOCK), lambda i: (i, 0, 0)),
            pl.BlockSpec((1, 1, _BLOCK), lambda i: (i, 0, 0)),
            pl.BlockSpec((T, _BLOCK), lambda i: (0, i)),
            pl.BlockSpec((T, _BLOCK), lambda i: (0, i)),
            pl.BlockSpec((T, _BLOCK), lambda i: (0, i)),
        ],
        out_specs=[
            pl.BlockSpec((_WIN - _MAX_TAU + max_tau + T, _BLOCK),
                         lambda i: (0, i)),
            pl.BlockSpec((1, 1, _BLOCK), lambda i: (i, 0, 0)),
        ],
        out_shape=[
            jax.ShapeDtypeStruct((_WIN - _MAX_TAU + max_tau + T, N_static),
                                 jnp.float32),
            jax.ShapeDtypeStruct((nb, 1, _BLOCK), jnp.float32),
        ],
        compiler_params=pltpu.CompilerParams(
            dimension_semantics=("arbitrary",)),
    )(scal, sig0, tau_r, lp0_r, et_T, ep_T, el_T)

    zs = zsT[_WIN - _MAX_TAU:, :].T
    xs = jax.lax.stop_gradient(zs[:, -T:])
    logp0 = lp_out.reshape(N_static)
    nat = jnp.zeros((N_static, T * 2), dtype=xs.dtype)
    return (tau, et, epsilont, zs, xs, -logp0, nat)
